# baseline (speedup 1.0000x reference)
import functools

import jax
import jax.numpy as jnp
from jax import lax
from jax.experimental import pallas as pl
from jax.experimental.pallas import tpu as pltpu
from jax.experimental.pallas import tpu_sc as plsc

B, T, C = 16, 2048, 512
G, V, VD = 2, 320, 256
N = B * T
P = N * G
BLK = 2048

NW = 32
PPW = P // NW
CHUNK = 128
NCHUNK = PPW // CHUNK


def _tc_body(x_ref, gu_ref, W_ref, b_ref, idx_ref):
    logits = jnp.dot(x_ref[...], W_ref[...], preferred_element_type=jnp.float32)
    logits = logits + b_ref[...]
    eps = 1e-10
    u = gu_ref[...] * (1.0 - 2.0 * eps) + eps
    y = logits - jnp.log(-jnp.log(u))
    for g in range(G):
        yg = y[:, g * V:(g + 1) * V]
        m = jnp.max(yg, axis=1, keepdims=True)
        iota = lax.broadcasted_iota(jnp.int32, (BLK, V), 1)
        first = jnp.min(jnp.where(yg == m, iota, V), axis=1, keepdims=True)
        idx_ref[:, g:g + 1] = first + g * V


def _sc_gather(tbl_hbm, idx_hbm, out_hbm, idx_v, buf0, buf1, sem0, sem1):
    wid = lax.axis_index("s") * 2 + lax.axis_index("c")
    base = wid * PPW
    pltpu.sync_copy(idx_hbm.at[wid], idx_v)
    bufs = (buf0, buf1)
    sems = (sem0, sem1)
    cps = [pltpu.async_copy(tbl_hbm.at[idx_v.at[0]], buf0, sem0), None]
    for c in range(NCHUNK):
        nxt = c + 1
        if nxt < NCHUNK:
            cps[nxt % 2] = pltpu.async_copy(
                tbl_hbm.at[idx_v.at[nxt]], bufs[nxt % 2], sems[nxt % 2])
        cps[c % 2].wait()
        pltpu.sync_copy(bufs[c % 2],
                        out_hbm.at[pl.ds(base + c * CHUNK, CHUNK)])


def kernel(x, gumbel_u, W, b, codevectors):
    x2 = x.reshape(N, C)
    gu2 = gumbel_u.reshape(N, G * V)
    grid = (N // BLK,)
    idx = pl.pallas_call(
        _tc_body,
        grid=grid,
        in_specs=[
            pl.BlockSpec((BLK, C), lambda i: (i, 0)),
            pl.BlockSpec((BLK, G * V), lambda i: (i, 0)),
            pl.BlockSpec((C, G * V), lambda i: (0, 0)),
            pl.BlockSpec((1, G * V), lambda i: (0, 0)),
        ],
        out_specs=pl.BlockSpec((BLK, G), lambda i: (i, 0)),
        out_shape=jax.ShapeDtypeStruct((N, G), jnp.int32),
    )(x2, gu2, W, b.reshape(1, G * V))

    idx3 = idx.reshape(NW, NCHUNK, CHUNK)

    mesh = plsc.VectorSubcoreMesh(core_axis_name="c", subcore_axis_name="s")
    gathered = pl.kernel(
        _sc_gather,
        out_type=jax.ShapeDtypeStruct((P, VD), jnp.float32),
        mesh=mesh,
        scratch_types=[
            pltpu.VMEM((NCHUNK, CHUNK), jnp.int32),
            pltpu.VMEM((CHUNK, VD), jnp.float32),
            pltpu.VMEM((CHUNK, VD), jnp.float32),
            pltpu.SemaphoreType.DMA,
            pltpu.SemaphoreType.DMA,
        ],
    )(codevectors, idx3)

    return gathered.reshape(B, T, G * VD)


# speedup vs baseline: 2.3033x; 2.3033x over previous
import jax
import jax.numpy as jnp
from jax import lax
from jax.experimental import pallas as pl

B, T, C = 16, 2048, 512
G, V, VD = 2, 320, 256
N = B * T
BLK = 2048


def _body(x_ref, gu_ref, W_ref, b_ref, cv_ref, out_ref):
    x = x_ref[...]
    logits = jnp.dot(x, W_ref[...], preferred_element_type=jnp.float32)
    logits = logits + b_ref[...]
    eps = 1e-10
    u = gu_ref[...] * (1.0 - 2.0 * eps) + eps
    y = logits - jnp.log(-jnp.log(u))
    for g in range(G):
        yg = y[:, g * V:(g + 1) * V]
        m = jnp.max(yg, axis=1, keepdims=True)
        oh = (yg == m).astype(jnp.float32)
        qg = jnp.dot(oh, cv_ref[g], preferred_element_type=jnp.float32)
        out_ref[:, g * VD:(g + 1) * VD] = qg


def kernel(x, gumbel_u, W, b, codevectors):
    x2 = x.reshape(N, C)
    gu2 = gumbel_u.reshape(N, G * V)
    cv3 = codevectors.reshape(G, V, VD)
    grid = (N // BLK,)
    out = pl.pallas_call(
        _body,
        grid=grid,
        in_specs=[
            pl.BlockSpec((BLK, C), lambda i: (i, 0)),
            pl.BlockSpec((BLK, G * V), lambda i: (i, 0)),
            pl.BlockSpec((C, G * V), lambda i: (0, 0)),
            pl.BlockSpec((1, G * V), lambda i: (0, 0)),
            pl.BlockSpec((G, V, VD), lambda i: (0, 0, 0)),
        ],
        out_specs=pl.BlockSpec((BLK, G * VD), lambda i: (i, 0)),
        out_shape=jax.ShapeDtypeStruct((N, G * VD), jnp.float32),
    )(x2, gu2, W, b.reshape(1, G * V), cv3)
    return out.reshape(B, T, G * VD)


# speedup vs baseline: 2.7053x; 1.1746x over previous
import jax
import jax.numpy as jnp
from jax.experimental import pallas as pl
from jax.experimental.pallas import tpu as pltpu

B, T, C = 16, 2048, 512
G, V, VD = 2, 320, 256
N = B * T
BLK = 2048


def _body(x_ref, gu_ref, W_ref, b_ref, cv_ref, out_ref):
    x = x_ref[...]
    logits = jnp.dot(x, W_ref[...], preferred_element_type=jnp.float32)
    logits = logits + b_ref[...]
    eps = 1e-10
    u = gu_ref[...] * (1.0 - 2.0 * eps) + eps
    y = logits - jnp.log(-jnp.log(u))
    for g in range(G):
        yg = y[:, g * V:(g + 1) * V]
        m = jnp.max(yg, axis=1, keepdims=True)
        iota = jax.lax.broadcasted_iota(jnp.int32, (BLK, V), 1)
        first = jnp.min(jnp.where(yg == m, iota, V), axis=1, keepdims=True)
        oh = (iota == first).astype(jnp.float32)
        qg = jnp.dot(oh, cv_ref[g], preferred_element_type=jnp.float32)
        out_ref[:, g * VD:(g + 1) * VD] = qg


def kernel(x, gumbel_u, W, b, codevectors):
    x2 = x.reshape(N, C)
    gu2 = gumbel_u.reshape(N, G * V)
    cv3 = codevectors.reshape(G, V, VD)
    grid = (N // BLK,)
    out = pl.pallas_call(
        _body,
        grid=grid,
        in_specs=[
            pl.BlockSpec((BLK, C), lambda i: (i, 0)),
            pl.BlockSpec((BLK, G * V), lambda i: (i, 0)),
            pl.BlockSpec((C, G * V), lambda i: (0, 0)),
            pl.BlockSpec((1, G * V), lambda i: (0, 0)),
            pl.BlockSpec((G, V, VD), lambda i: (0, 0, 0)),
        ],
        out_specs=pl.BlockSpec((BLK, G * VD), lambda i: (i, 0)),
        out_shape=jax.ShapeDtypeStruct((N, G * VD), jnp.float32),
    )(x2, gu2, W, b.reshape(1, G * V), cv3)
    return out.reshape(B, T, G * VD)
